# baseline (speedup 1.0000x reference)
import jax
import jax.numpy as jnp
from jax import lax
from jax.experimental import pallas as pl
from jax.experimental.pallas import tpu as pltpu
from jax.experimental.pallas import tpu_sc as plsc

L = 16
NC = 2
NS = 16
NW = NC * NS


def _eg_body(entt_ref, gatet_ref, eg_ref):
    et = entt_ref[...].T
    D = et.shape[1]
    eg_ref[:, :D] = et
    eg_ref[:, D:] = gatet_ref[...].T


def _eg_call(entt, gatet, bn=2048):
    D, N = entt.shape
    return pl.pallas_call(
        _eg_body,
        grid=((N + bn - 1) // bn,),
        in_specs=[
            pl.BlockSpec((D, bn), lambda i: (0, i)),
            pl.BlockSpec((D, bn), lambda i: (0, i)),
        ],
        out_specs=pl.BlockSpec((bn, 2 * D), lambda i: (i, 0)),
        out_shape=jax.ShapeDtypeStruct((N, 2 * D), jnp.float32),
    )(entt, gatet)


def _z_body(wordt_ref, z_ref):
    D = wordt_ref.shape[0]
    z_ref[:, :D] = wordt_ref[...].T


def _z_call(wordt, n, bn=2048):
    D = wordt.shape[0]
    return pl.pallas_call(
        _z_body,
        grid=((n + bn - 1) // bn,),
        in_specs=[pl.BlockSpec((D, bn), lambda i: (0, i))],
        out_specs=pl.BlockSpec((bn, 2 * D), lambda i: (i, 0)),
        out_shape=jax.ShapeDtypeStruct((n, 2 * D), jnp.float32),
    )(wordt)


def _make_sc_eg(B, D2):
    RPW = B // NW

    def body(head_hbm, tail_hbm, eg_hbm, heg_hbm, teg_hbm,
             hidx, tidx, hrows, trows, esem):
        wid = lax.axis_index("s") * NC + lax.axis_index("c")
        base = wid * RPW
        pltpu.sync_copy(head_hbm.at[pl.ds(base, RPW)], hidx)
        pltpu.sync_copy(tail_hbm.at[pl.ds(base, RPW)], tidx)
        cp_h = pltpu.async_copy(eg_hbm.at[hidx], hrows, esem)
        cp_t = pltpu.async_copy(eg_hbm.at[tidx], trows, esem)
        cp_h.wait()
        cp_t.wait()
        pltpu.sync_copy(hrows, heg_hbm.at[pl.ds(base, RPW)])
        pltpu.sync_copy(trows, teg_hbm.at[pl.ds(base, RPW)])

    return pl.kernel(
        body,
        out_type=[jax.ShapeDtypeStruct((B, D2), jnp.float32),
                  jax.ShapeDtypeStruct((B, D2), jnp.float32)],
        mesh=plsc.VectorSubcoreMesh(core_axis_name="c", subcore_axis_name="s"),
        compiler_params=pltpu.CompilerParams(use_tc_tiling_on_sc=False),
        scratch_types=[
            pltpu.VMEM((RPW,), jnp.int32),
            pltpu.VMEM((RPW,), jnp.int32),
            pltpu.VMEM((RPW, D2), jnp.float32),
            pltpu.VMEM((RPW, D2), jnp.float32),
            pltpu.SemaphoreType.DMA,
        ],
    )


def _make_sc_wordsum(B, D, W):
    RPW = B // NW
    SPC = 2
    IDXM = SPC * W
    CPW = RPW // SPC
    NBUF = 4

    def body(hd_hbm, td_hbm, word_hbm, wsum_hbm,
             widx, acc, *bufs_sems):
        bufs = bufs_sems[:NBUF]
        sems = bufs_sems[NBUF:NBUF + NBUF]
        wid = lax.axis_index("s") * NC + lax.axis_index("c")
        base = wid * RPW

        def word_phase(src_hbm, col0):
            pltpu.sync_copy(src_hbm.at[pl.ds(wid * CPW, CPW)], widx)
            for p in range(NBUF - 1):
                pltpu.async_copy(word_hbm.at[widx.at[p]], bufs[p], sems[p])

            def g_body(g, _):
                for b in range(NBUF):
                    c = g * NBUF + b
                    buf, sem = bufs[b], sems[b]
                    nb = (b + NBUF - 1) % NBUF
                    pltpu.make_async_copy(
                        word_hbm.at[widx.at[c]], buf, sem).wait()

                    @pl.when(c + NBUF - 1 < CPW)
                    def _issue_next():
                        pltpu.async_copy(
                            word_hbm.at[widx.at[c + NBUF - 1]],
                            bufs[nb], sems[nb])

                    for s in range(SPC):
                        def j_body(j, accs, _s=s):
                            return tuple(
                                accs[k] + buf[_s * W + j, pl.ds(k * L, L)]
                                for k in range(D // L))
                        accs = lax.fori_loop(
                            0, W, j_body,
                            tuple(jnp.zeros((L,), jnp.float32)
                                  for _ in range(D // L)))
                        row = SPC * c + s
                        for k in range(D // L):
                            acc[row, pl.ds(col0 + k * L, L)] = accs[k]
                return 0

            lax.fori_loop(0, CPW // NBUF, g_body, 0)

        word_phase(hd_hbm, 0)
        word_phase(td_hbm, D)
        pltpu.sync_copy(acc, wsum_hbm.at[pl.ds(base, RPW)])

    return pl.kernel(
        body,
        out_type=[jax.ShapeDtypeStruct((B, 2 * D), jnp.float32)],
        mesh=plsc.VectorSubcoreMesh(core_axis_name="c", subcore_axis_name="s"),
        compiler_params=pltpu.CompilerParams(use_tc_tiling_on_sc=False),
        scratch_types=[
            pltpu.VMEM((CPW, IDXM), jnp.int32),
            pltpu.VMEM((RPW, 2 * D), jnp.float32),
        ] + [pltpu.VMEM((IDXM, D), jnp.float32) for _ in range(NBUF)]
          + [pltpu.SemaphoreType.DMA for _ in range(NBUF)],
    )


def _dense_body(heg_ref, teg_ref, ws_ref, wh_ref, wt_ref, wb_ref,
                relt_ref, negt_ref, ot_ref):
    D = wh_ref.shape[1]
    heg = heg_ref[...]
    teg = teg_ref[...]
    ws = ws_ref[...]
    gh = jax.nn.sigmoid(heg[:, D:])
    gt = jax.nn.sigmoid(teg[:, D:])
    h = gh * heg[:, :D] + (1.0 - gh) * ws[:, :D]
    t = gt * teg[:, :D] + (1.0 - gt) * ws[:, D:]
    hrt = jnp.tanh(h * wh_ref[...] + t * wt_ref[...] + wb_ref[...])
    scores = lax.dot_general(relt_ref[...], hrt, (((0,), (1,)), ((), ())),
                             preferred_element_type=jnp.float32)
    w = jnp.abs(negt_ref[...])
    m = jnp.max(w * scores, axis=0, keepdims=True)
    e = jnp.exp(scores - m)
    s = jnp.sum(e * w, axis=0, keepdims=True)
    ot_ref[...] = e / s * w


def _dense_call(heg, teg, wsum, wh, wt, wb, relt, negt, bb=512):
    B = heg.shape[0]
    D = wh.shape[1]
    R = negt.shape[0]
    return pl.pallas_call(
        _dense_body,
        grid=(B // bb,),
        in_specs=[
            pl.BlockSpec((bb, 2 * D), lambda i: (i, 0)),
            pl.BlockSpec((bb, 2 * D), lambda i: (i, 0)),
            pl.BlockSpec((bb, 2 * D), lambda i: (i, 0)),
            pl.BlockSpec((1, D), lambda i: (0, 0)),
            pl.BlockSpec((1, D), lambda i: (0, 0)),
            pl.BlockSpec((1, D), lambda i: (0, 0)),
            pl.BlockSpec((D, R), lambda i: (0, 0)),
            pl.BlockSpec((R, bb), lambda i: (0, i)),
        ],
        out_specs=pl.BlockSpec((R, bb), lambda i: (0, i)),
        out_shape=jax.ShapeDtypeStruct((R, B), jnp.float32),
    )(heg, teg, wsum, wh, wt, wb, relt, negt)


def kernel(triple, hd, td, neg_sample_r, entity_emb, relation_emb, word_emb,
           gate_emb, weight_h, weight_t, weight_bias):
    B, W = hd.shape
    D = entity_emb.shape[1]
    V = entity_emb.shape[0]
    head = triple[:, 0]
    tail = triple[:, 1]
    z = _z_call(word_emb.T, V)
    word_lin = z.reshape(2 * V, D)
    hd2 = (hd * 2).reshape(B * W // (2 * W), 2 * W)
    td2 = (td * 2).reshape(B * W // (2 * W), 2 * W)
    (wsum,) = _make_sc_wordsum(B, D, W)(hd2, td2, word_lin)
    eg = _eg_call(entity_emb.T, gate_emb.T)
    heg, teg = _make_sc_eg(B, 2 * D)(head, tail, eg)
    out_t = _dense_call(heg, teg, wsum, weight_h.reshape(1, D),
                        weight_t.reshape(1, D), weight_bias.reshape(1, D),
                        relation_emb.T, neg_sample_r.T)
    return out_t.T


# speedup vs baseline: 1.1972x; 1.1972x over previous
import jax
import jax.numpy as jnp
from jax import lax
from jax.experimental import pallas as pl
from jax.experimental.pallas import tpu as pltpu
from jax.experimental.pallas import tpu_sc as plsc

L = 16
NC = 2
NS = 16
NW = NC * NS


def _eg_body(entt_ref, gatet_ref, eg_ref):
    et = entt_ref[...].T
    D = et.shape[1]
    eg_ref[:, :D] = et
    eg_ref[:, D:] = gatet_ref[...].T


def _eg_call(entt, gatet, bn=2048):
    D, N = entt.shape
    return pl.pallas_call(
        _eg_body,
        grid=((N + bn - 1) // bn,),
        in_specs=[
            pl.BlockSpec((D, bn), lambda i: (0, i)),
            pl.BlockSpec((D, bn), lambda i: (0, i)),
        ],
        out_specs=pl.BlockSpec((bn, 2 * D), lambda i: (i, 0)),
        out_shape=jax.ShapeDtypeStruct((N, 2 * D), jnp.float32),
    )(entt, gatet)


def _z_body(wordt_ref, z_ref):
    D = wordt_ref.shape[0]
    z_ref[:, :D] = wordt_ref[...].T


def _z_call(wordt, n, bn=2048):
    D = wordt.shape[0]
    return pl.pallas_call(
        _z_body,
        grid=((n + bn - 1) // bn,),
        in_specs=[pl.BlockSpec((D, bn), lambda i: (0, i))],
        out_specs=pl.BlockSpec((bn, 2 * D), lambda i: (i, 0)),
        out_shape=jax.ShapeDtypeStruct((n, 2 * D), jnp.float32),
    )(wordt)


def _make_sc_eg(B, D2):
    RPW = B // NW

    def body(head_hbm, tail_hbm, eg_hbm, heg_hbm, teg_hbm,
             hidx, tidx, hrows, trows, esem):
        wid = lax.axis_index("s") * NC + lax.axis_index("c")
        base = wid * RPW
        pltpu.sync_copy(head_hbm.at[pl.ds(base, RPW)], hidx)
        pltpu.sync_copy(tail_hbm.at[pl.ds(base, RPW)], tidx)
        cp_h = pltpu.async_copy(eg_hbm.at[hidx], hrows, esem)
        cp_t = pltpu.async_copy(eg_hbm.at[tidx], trows, esem)
        cp_h.wait()
        cp_t.wait()
        pltpu.sync_copy(hrows, heg_hbm.at[pl.ds(base, RPW)])
        pltpu.sync_copy(trows, teg_hbm.at[pl.ds(base, RPW)])

    return pl.kernel(
        body,
        out_type=[jax.ShapeDtypeStruct((B, D2), jnp.float32),
                  jax.ShapeDtypeStruct((B, D2), jnp.float32)],
        mesh=plsc.VectorSubcoreMesh(core_axis_name="c", subcore_axis_name="s"),
        compiler_params=pltpu.CompilerParams(use_tc_tiling_on_sc=False),
        scratch_types=[
            pltpu.VMEM((RPW,), jnp.int32),
            pltpu.VMEM((RPW,), jnp.int32),
            pltpu.VMEM((RPW, D2), jnp.float32),
            pltpu.VMEM((RPW, D2), jnp.float32),
            pltpu.SemaphoreType.DMA,
        ],
    )


def _make_sc_wordsum(B, D, W):
    RPW = B // NW
    SPC = 2
    IDXM = SPC * W
    CPW = RPW // SPC
    NBUF = 8

    def body(hd_hbm, td_hbm, word_hbm, wsum_hbm,
             widx, acc, *bufs_sems):
        bufs = bufs_sems[:NBUF]
        sems = bufs_sems[NBUF:NBUF + NBUF]
        wid = lax.axis_index("s") * NC + lax.axis_index("c")
        base = wid * RPW

        def word_phase(src_hbm, col0):
            pltpu.sync_copy(src_hbm.at[pl.ds(wid * CPW, CPW)], widx)
            for p in range(NBUF - 1):
                pltpu.async_copy(word_hbm.at[widx.at[p]], bufs[p], sems[p])

            def g_body(g, _):
                for b in range(NBUF):
                    c = g * NBUF + b
                    buf, sem = bufs[b], sems[b]
                    nb = (b + NBUF - 1) % NBUF
                    pltpu.make_async_copy(
                        word_hbm.at[widx.at[c]], buf, sem).wait()

                    @pl.when(c + NBUF - 1 < CPW)
                    def _issue_next():
                        pltpu.async_copy(
                            word_hbm.at[widx.at[c + NBUF - 1]],
                            bufs[nb], sems[nb])

                    for s in range(SPC):
                        def j_body(j, accs, _s=s):
                            return tuple(
                                accs[k] + buf[_s * W + j, pl.ds(k * L, L)]
                                for k in range(D // L))
                        accs = lax.fori_loop(
                            0, W, j_body,
                            tuple(jnp.zeros((L,), jnp.float32)
                                  for _ in range(D // L)))
                        row = SPC * c + s
                        for k in range(D // L):
                            acc[row, pl.ds(col0 + k * L, L)] = accs[k]
                return 0

            lax.fori_loop(0, CPW // NBUF, g_body, 0)

        word_phase(hd_hbm, 0)
        word_phase(td_hbm, D)
        pltpu.sync_copy(acc, wsum_hbm.at[pl.ds(base, RPW)])

    return pl.kernel(
        body,
        out_type=[jax.ShapeDtypeStruct((B, 2 * D), jnp.float32)],
        mesh=plsc.VectorSubcoreMesh(core_axis_name="c", subcore_axis_name="s"),
        compiler_params=pltpu.CompilerParams(use_tc_tiling_on_sc=False),
        scratch_types=[
            pltpu.VMEM((CPW, IDXM), jnp.int32),
            pltpu.VMEM((RPW, 2 * D), jnp.float32),
        ] + [pltpu.VMEM((IDXM, D), jnp.float32) for _ in range(NBUF)]
          + [pltpu.SemaphoreType.DMA for _ in range(NBUF)],
    )


def _dense_body(heg_ref, teg_ref, ws_ref, wh_ref, wt_ref, wb_ref,
                relt_ref, negt_ref, ot_ref):
    D = wh_ref.shape[1]
    heg = heg_ref[...]
    teg = teg_ref[...]
    ws = ws_ref[...]
    gh = jax.nn.sigmoid(heg[:, D:])
    gt = jax.nn.sigmoid(teg[:, D:])
    h = gh * heg[:, :D] + (1.0 - gh) * ws[:, :D]
    t = gt * teg[:, :D] + (1.0 - gt) * ws[:, D:]
    hrt = jnp.tanh(h * wh_ref[...] + t * wt_ref[...] + wb_ref[...])
    scores = lax.dot_general(relt_ref[...], hrt, (((0,), (1,)), ((), ())),
                             preferred_element_type=jnp.float32)
    w = jnp.abs(negt_ref[...])
    m = jnp.max(w * scores, axis=0, keepdims=True)
    e = jnp.exp(scores - m)
    s = jnp.sum(e * w, axis=0, keepdims=True)
    ot_ref[...] = e / s * w


def _dense_call(heg, teg, wsum, wh, wt, wb, relt, negt, bb=512):
    B = heg.shape[0]
    D = wh.shape[1]
    R = negt.shape[0]
    return pl.pallas_call(
        _dense_body,
        grid=(B // bb,),
        in_specs=[
            pl.BlockSpec((bb, 2 * D), lambda i: (i, 0)),
            pl.BlockSpec((bb, 2 * D), lambda i: (i, 0)),
            pl.BlockSpec((bb, 2 * D), lambda i: (i, 0)),
            pl.BlockSpec((1, D), lambda i: (0, 0)),
            pl.BlockSpec((1, D), lambda i: (0, 0)),
            pl.BlockSpec((1, D), lambda i: (0, 0)),
            pl.BlockSpec((D, R), lambda i: (0, 0)),
            pl.BlockSpec((R, bb), lambda i: (0, i)),
        ],
        out_specs=pl.BlockSpec((R, bb), lambda i: (0, i)),
        out_shape=jax.ShapeDtypeStruct((R, B), jnp.float32),
    )(heg, teg, wsum, wh, wt, wb, relt, negt)


def kernel(triple, hd, td, neg_sample_r, entity_emb, relation_emb, word_emb,
           gate_emb, weight_h, weight_t, weight_bias):
    B, W = hd.shape
    D = entity_emb.shape[1]
    V = entity_emb.shape[0]
    head = triple[:, 0]
    tail = triple[:, 1]
    z = _z_call(word_emb.T, V)
    entt_b, gatet_b, z_b = lax.optimization_barrier(
        (entity_emb.T, gate_emb.T, z))
    word_lin = z_b.reshape(2 * V, D)
    hd2 = (hd * 2).reshape(B * W // (2 * W), 2 * W)
    td2 = (td * 2).reshape(B * W // (2 * W), 2 * W)
    (wsum,) = _make_sc_wordsum(B, D, W)(hd2, td2, word_lin)
    eg = _eg_call(entt_b, gatet_b)
    heg, teg = _make_sc_eg(B, 2 * D)(head, tail, eg)
    out_t = _dense_call(heg, teg, wsum, weight_h.reshape(1, D),
                        weight_t.reshape(1, D), weight_bias.reshape(1, D),
                        relation_emb.T, neg_sample_r.T)
    return out_t.T


# speedup vs baseline: 1.3300x; 1.1109x over previous
import jax
import jax.numpy as jnp
from jax import lax
from jax.experimental import pallas as pl
from jax.experimental.pallas import tpu as pltpu
from jax.experimental.pallas import tpu_sc as plsc

L = 16
NC = 2
NS = 16
NW = NC * NS


def _eg_body(entt_ref, gatet_ref, eg_ref):
    et = entt_ref[...].T
    D = et.shape[1]
    eg_ref[:, :D] = et
    eg_ref[:, D:] = gatet_ref[...].T


def _eg_call(entt, gatet, bn=4096):
    D, N = entt.shape
    return pl.pallas_call(
        _eg_body,
        grid=((N + bn - 1) // bn,),
        in_specs=[
            pl.BlockSpec((D, bn), lambda i: (0, i)),
            pl.BlockSpec((D, bn), lambda i: (0, i)),
        ],
        out_specs=pl.BlockSpec((bn, 2 * D), lambda i: (i, 0)),
        out_shape=jax.ShapeDtypeStruct((N, 2 * D), jnp.float32),
    )(entt, gatet)


def _z_body(wordt_ref, z_ref):
    D = wordt_ref.shape[0]
    z_ref[:, :D] = wordt_ref[...].T


def _z_call(wordt, n, bn=4096):
    D = wordt.shape[0]
    return pl.pallas_call(
        _z_body,
        grid=((n + bn - 1) // bn,),
        in_specs=[pl.BlockSpec((D, bn), lambda i: (0, i))],
        out_specs=pl.BlockSpec((bn, 2 * D), lambda i: (i, 0)),
        out_shape=jax.ShapeDtypeStruct((n, 2 * D), jnp.float32),
    )(wordt)


def _make_sc_eg(B, D2):
    RPW = B // NW

    def body(head_hbm, tail_hbm, eg_hbm, heg_hbm, teg_hbm,
             hidx, tidx, hrows, trows, esem):
        wid = lax.axis_index("s") * NC + lax.axis_index("c")
        base = wid * RPW
        pltpu.sync_copy(head_hbm.at[pl.ds(base, RPW)], hidx)
        pltpu.sync_copy(tail_hbm.at[pl.ds(base, RPW)], tidx)
        cp_h = pltpu.async_copy(eg_hbm.at[hidx], hrows, esem)
        cp_t = pltpu.async_copy(eg_hbm.at[tidx], trows, esem)
        cp_h.wait()
        cp_t.wait()
        pltpu.sync_copy(hrows, heg_hbm.at[pl.ds(base, RPW)])
        pltpu.sync_copy(trows, teg_hbm.at[pl.ds(base, RPW)])

    return pl.kernel(
        body,
        out_type=[jax.ShapeDtypeStruct((B, D2), jnp.float32),
                  jax.ShapeDtypeStruct((B, D2), jnp.float32)],
        mesh=plsc.VectorSubcoreMesh(core_axis_name="c", subcore_axis_name="s"),
        compiler_params=pltpu.CompilerParams(use_tc_tiling_on_sc=False),
        scratch_types=[
            pltpu.VMEM((RPW,), jnp.int32),
            pltpu.VMEM((RPW,), jnp.int32),
            pltpu.VMEM((RPW, D2), jnp.float32),
            pltpu.VMEM((RPW, D2), jnp.float32),
            pltpu.SemaphoreType.DMA,
        ],
    )


def _make_sc_wordsum(B, D, W):
    RPW = B // NW
    SPC = 2
    IDXM = SPC * W
    CPW = RPW // SPC
    NBUF = 8

    def body(hd_hbm, td_hbm, word_hbm, wsum_hbm,
             widx, acc, *bufs_sems):
        bufs = bufs_sems[:NBUF]
        sems = bufs_sems[NBUF:NBUF + NBUF]
        wid = lax.axis_index("s") * NC + lax.axis_index("c")
        base = wid * RPW

        def word_phase(src_hbm, col0):
            pltpu.sync_copy(src_hbm.at[pl.ds(wid * CPW, CPW)], widx)
            for p in range(NBUF - 1):
                pltpu.async_copy(word_hbm.at[widx.at[p]], bufs[p], sems[p])

            def g_body(g, _):
                for b in range(NBUF):
                    c = g * NBUF + b
                    buf, sem = bufs[b], sems[b]
                    nb = (b + NBUF - 1) % NBUF
                    pltpu.make_async_copy(
                        word_hbm.at[widx.at[c]], buf, sem).wait()

                    @pl.when(c + NBUF - 1 < CPW)
                    def _issue_next():
                        pltpu.async_copy(
                            word_hbm.at[widx.at[c + NBUF - 1]],
                            bufs[nb], sems[nb])

                    for s in range(SPC):
                        def j_body(j, accs, _s=s):
                            return tuple(
                                accs[k] + buf[_s * W + j, pl.ds(k * L, L)]
                                for k in range(D // L))
                        accs = lax.fori_loop(
                            0, W, j_body,
                            tuple(jnp.zeros((L,), jnp.float32)
                                  for _ in range(D // L)))
                        row = SPC * c + s
                        for k in range(D // L):
                            acc[row, pl.ds(col0 + k * L, L)] = accs[k]
                return 0

            lax.fori_loop(0, CPW // NBUF, g_body, 0)

        word_phase(hd_hbm, 0)
        word_phase(td_hbm, D)
        pltpu.sync_copy(acc, wsum_hbm.at[pl.ds(base, RPW)])

    return pl.kernel(
        body,
        out_type=[jax.ShapeDtypeStruct((B, 2 * D), jnp.float32)],
        mesh=plsc.VectorSubcoreMesh(core_axis_name="c", subcore_axis_name="s"),
        compiler_params=pltpu.CompilerParams(use_tc_tiling_on_sc=False),
        scratch_types=[
            pltpu.VMEM((CPW, IDXM), jnp.int32),
            pltpu.VMEM((RPW, 2 * D), jnp.float32),
        ] + [pltpu.VMEM((IDXM, D), jnp.float32) for _ in range(NBUF)]
          + [pltpu.SemaphoreType.DMA for _ in range(NBUF)],
    )


def _dense_body(heg_ref, teg_ref, ws_ref, wh_ref, wt_ref, wb_ref,
                relt_ref, negt_ref, ot_ref):
    D = wh_ref.shape[1]
    heg = heg_ref[...]
    teg = teg_ref[...]
    ws = ws_ref[...]
    gh = jax.nn.sigmoid(heg[:, D:])
    gt = jax.nn.sigmoid(teg[:, D:])
    h = gh * heg[:, :D] + (1.0 - gh) * ws[:, :D]
    t = gt * teg[:, :D] + (1.0 - gt) * ws[:, D:]
    hrt = jnp.tanh(h * wh_ref[...] + t * wt_ref[...] + wb_ref[...])
    scores = lax.dot_general(relt_ref[...], hrt, (((0,), (1,)), ((), ())),
                             preferred_element_type=jnp.float32)
    w = jnp.abs(negt_ref[...])
    m = jnp.max(w * scores, axis=0, keepdims=True)
    e = jnp.exp(scores - m)
    s = jnp.sum(e * w, axis=0, keepdims=True)
    ot_ref[...] = e / s * w


def _dense_call(heg, teg, wsum, wh, wt, wb, relt, negt, bb=1024):
    B = heg.shape[0]
    D = wh.shape[1]
    R = negt.shape[0]
    return pl.pallas_call(
        _dense_body,
        grid=(B // bb,),
        in_specs=[
            pl.BlockSpec((bb, 2 * D), lambda i: (i, 0)),
            pl.BlockSpec((bb, 2 * D), lambda i: (i, 0)),
            pl.BlockSpec((bb, 2 * D), lambda i: (i, 0)),
            pl.BlockSpec((1, D), lambda i: (0, 0)),
            pl.BlockSpec((1, D), lambda i: (0, 0)),
            pl.BlockSpec((1, D), lambda i: (0, 0)),
            pl.BlockSpec((D, R), lambda i: (0, 0)),
            pl.BlockSpec((R, bb), lambda i: (0, i)),
        ],
        out_specs=pl.BlockSpec((R, bb), lambda i: (0, i)),
        out_shape=jax.ShapeDtypeStruct((R, B), jnp.float32),
    )(heg, teg, wsum, wh, wt, wb, relt, negt)


def kernel(triple, hd, td, neg_sample_r, entity_emb, relation_emb, word_emb,
           gate_emb, weight_h, weight_t, weight_bias):
    B, W = hd.shape
    D = entity_emb.shape[1]
    V = entity_emb.shape[0]
    head = triple[:, 0]
    tail = triple[:, 1]
    z = _z_call(word_emb.T, V)
    entt_b, gatet_b, z_b = lax.optimization_barrier(
        (entity_emb.T, gate_emb.T, z))
    word_lin = z_b.reshape(2 * V, D)
    hd2 = (hd * 2).reshape(B * W // (2 * W), 2 * W)
    td2 = (td * 2).reshape(B * W // (2 * W), 2 * W)
    (wsum,) = _make_sc_wordsum(B, D, W)(hd2, td2, word_lin)
    eg = _eg_call(entt_b, gatet_b)
    heg, teg = _make_sc_eg(B, 2 * D)(head, tail, eg)
    out_t = _dense_call(heg, teg, wsum, weight_h.reshape(1, D),
                        weight_t.reshape(1, D), weight_bias.reshape(1, D),
                        relation_emb.T, neg_sample_r.T)
    return out_t.T


# speedup vs baseline: 1.3723x; 1.0319x over previous
import jax
import jax.numpy as jnp
from jax import lax
from jax.experimental import pallas as pl
from jax.experimental.pallas import tpu as pltpu
from jax.experimental.pallas import tpu_sc as plsc

L = 16
NC = 2
NS = 16
NW = NC * NS


def _eg_body(entt_ref, gatet_ref, eg_ref):
    et = entt_ref[...].T
    D = et.shape[1]
    eg_ref[:, :D] = et
    eg_ref[:, D:] = gatet_ref[...].T


def _eg_call(entt, gatet, bn=8192):
    D, N = entt.shape
    return pl.pallas_call(
        _eg_body,
        grid=((N + bn - 1) // bn,),
        in_specs=[
            pl.BlockSpec((D, bn), lambda i: (0, i)),
            pl.BlockSpec((D, bn), lambda i: (0, i)),
        ],
        out_specs=pl.BlockSpec((bn, 2 * D), lambda i: (i, 0)),
        out_shape=jax.ShapeDtypeStruct((N, 2 * D), jnp.float32),
    )(entt, gatet)


def _z_body(wordt_ref, z_ref):
    D = wordt_ref.shape[0]
    z_ref[:, :D] = wordt_ref[...].T


def _z_call(wordt, n, bn=8192):
    D = wordt.shape[0]
    return pl.pallas_call(
        _z_body,
        grid=((n + bn - 1) // bn,),
        in_specs=[pl.BlockSpec((D, bn), lambda i: (0, i))],
        out_specs=pl.BlockSpec((bn, 2 * D), lambda i: (i, 0)),
        out_shape=jax.ShapeDtypeStruct((n, 2 * D), jnp.float32),
    )(wordt)


def _make_sc_eg(B, D2):
    RPW = B // NW

    def body(head_hbm, tail_hbm, eg_hbm, heg_hbm, teg_hbm,
             hidx, tidx, hrows, trows, esem):
        wid = lax.axis_index("s") * NC + lax.axis_index("c")
        base = wid * RPW
        pltpu.sync_copy(head_hbm.at[pl.ds(base, RPW)], hidx)
        pltpu.sync_copy(tail_hbm.at[pl.ds(base, RPW)], tidx)
        cp_h = pltpu.async_copy(eg_hbm.at[hidx], hrows, esem)
        cp_t = pltpu.async_copy(eg_hbm.at[tidx], trows, esem)
        cp_h.wait()
        cp_t.wait()
        pltpu.sync_copy(hrows, heg_hbm.at[pl.ds(base, RPW)])
        pltpu.sync_copy(trows, teg_hbm.at[pl.ds(base, RPW)])

    return pl.kernel(
        body,
        out_type=[jax.ShapeDtypeStruct((B, D2), jnp.float32),
                  jax.ShapeDtypeStruct((B, D2), jnp.float32)],
        mesh=plsc.VectorSubcoreMesh(core_axis_name="c", subcore_axis_name="s"),
        compiler_params=pltpu.CompilerParams(use_tc_tiling_on_sc=False),
        scratch_types=[
            pltpu.VMEM((RPW,), jnp.int32),
            pltpu.VMEM((RPW,), jnp.int32),
            pltpu.VMEM((RPW, D2), jnp.float32),
            pltpu.VMEM((RPW, D2), jnp.float32),
            pltpu.SemaphoreType.DMA,
        ],
    )


def _make_sc_wordsum(B, D, W):
    RPW = B // NW
    SPC = 2
    IDXM = SPC * W
    CPW = RPW // SPC
    NBUF = 8

    def body(hd_hbm, td_hbm, word_hbm, wsum_hbm,
             widx, acc, *bufs_sems):
        bufs = bufs_sems[:NBUF]
        sems = bufs_sems[NBUF:NBUF + NBUF]
        wid = lax.axis_index("s") * NC + lax.axis_index("c")
        base = wid * RPW

        def word_phase(src_hbm, col0):
            pltpu.sync_copy(src_hbm.at[pl.ds(wid * CPW, CPW)], widx)
            for p in range(NBUF - 1):
                pltpu.async_copy(word_hbm.at[widx.at[p]], bufs[p], sems[p])

            def g_body(g, _):
                for b in range(NBUF):
                    c = g * NBUF + b
                    buf, sem = bufs[b], sems[b]
                    nb = (b + NBUF - 1) % NBUF
                    pltpu.make_async_copy(
                        word_hbm.at[widx.at[c]], buf, sem).wait()

                    @pl.when(c + NBUF - 1 < CPW)
                    def _issue_next():
                        pltpu.async_copy(
                            word_hbm.at[widx.at[c + NBUF - 1]],
                            bufs[nb], sems[nb])

                    for s in range(SPC):
                        def j_body(j, accs, _s=s):
                            return tuple(
                                accs[k] + buf[_s * W + j, pl.ds(k * L, L)]
                                for k in range(D // L))
                        accs = lax.fori_loop(
                            0, W, j_body,
                            tuple(jnp.zeros((L,), jnp.float32)
                                  for _ in range(D // L)))
                        row = SPC * c + s
                        for k in range(D // L):
                            acc[row, pl.ds(col0 + k * L, L)] = accs[k]
                return 0

            lax.fori_loop(0, CPW // NBUF, g_body, 0)

        word_phase(hd_hbm, 0)
        word_phase(td_hbm, D)
        pltpu.sync_copy(acc, wsum_hbm.at[pl.ds(base, RPW)])

    return pl.kernel(
        body,
        out_type=[jax.ShapeDtypeStruct((B, 2 * D), jnp.float32)],
        mesh=plsc.VectorSubcoreMesh(core_axis_name="c", subcore_axis_name="s"),
        compiler_params=pltpu.CompilerParams(use_tc_tiling_on_sc=False),
        scratch_types=[
            pltpu.VMEM((CPW, IDXM), jnp.int32),
            pltpu.VMEM((RPW, 2 * D), jnp.float32),
        ] + [pltpu.VMEM((IDXM, D), jnp.float32) for _ in range(NBUF)]
          + [pltpu.SemaphoreType.DMA for _ in range(NBUF)],
    )


def _dense_body(heg_ref, teg_ref, ws_ref, wh_ref, wt_ref, wb_ref,
                relt_ref, negt_ref, ot_ref):
    D = wh_ref.shape[1]
    heg = heg_ref[...]
    teg = teg_ref[...]
    ws = ws_ref[...]
    gh = jax.nn.sigmoid(heg[:, D:])
    gt = jax.nn.sigmoid(teg[:, D:])
    h = gh * heg[:, :D] + (1.0 - gh) * ws[:, :D]
    t = gt * teg[:, :D] + (1.0 - gt) * ws[:, D:]
    hrt = jnp.tanh(h * wh_ref[...] + t * wt_ref[...] + wb_ref[...])
    scores = lax.dot_general(relt_ref[...], hrt, (((0,), (1,)), ((), ())),
                             preferred_element_type=jnp.float32)
    w = jnp.abs(negt_ref[...])
    m = jnp.max(w * scores, axis=0, keepdims=True)
    e = jnp.exp(scores - m)
    s = jnp.sum(e * w, axis=0, keepdims=True)
    ot_ref[...] = e / s * w


def _dense_call(heg, teg, wsum, wh, wt, wb, relt, negt, bb=2048):
    B = heg.shape[0]
    D = wh.shape[1]
    R = negt.shape[0]
    return pl.pallas_call(
        _dense_body,
        grid=(B // bb,),
        in_specs=[
            pl.BlockSpec((bb, 2 * D), lambda i: (i, 0)),
            pl.BlockSpec((bb, 2 * D), lambda i: (i, 0)),
            pl.BlockSpec((bb, 2 * D), lambda i: (i, 0)),
            pl.BlockSpec((1, D), lambda i: (0, 0)),
            pl.BlockSpec((1, D), lambda i: (0, 0)),
            pl.BlockSpec((1, D), lambda i: (0, 0)),
            pl.BlockSpec((D, R), lambda i: (0, 0)),
            pl.BlockSpec((R, bb), lambda i: (0, i)),
        ],
        out_specs=pl.BlockSpec((R, bb), lambda i: (0, i)),
        out_shape=jax.ShapeDtypeStruct((R, B), jnp.float32),
    )(heg, teg, wsum, wh, wt, wb, relt, negt)


def kernel(triple, hd, td, neg_sample_r, entity_emb, relation_emb, word_emb,
           gate_emb, weight_h, weight_t, weight_bias):
    B, W = hd.shape
    D = entity_emb.shape[1]
    V = entity_emb.shape[0]
    head = triple[:, 0]
    tail = triple[:, 1]
    z = _z_call(word_emb.T, V)
    entt_b, gatet_b, z_b = lax.optimization_barrier(
        (entity_emb.T, gate_emb.T, z))
    word_lin = z_b.reshape(2 * V, D)
    hd2 = (hd * 2).reshape(B * W // (2 * W), 2 * W)
    td2 = (td * 2).reshape(B * W // (2 * W), 2 * W)
    (wsum,) = _make_sc_wordsum(B, D, W)(hd2, td2, word_lin)
    eg = _eg_call(entt_b, gatet_b)
    heg, teg = _make_sc_eg(B, 2 * D)(head, tail, eg)
    out_t = _dense_call(heg, teg, wsum, weight_h.reshape(1, D),
                        weight_t.reshape(1, D), weight_bias.reshape(1, D),
                        relation_emb.T, neg_sample_r.T)
    return out_t.T


# speedup vs baseline: 1.3788x; 1.0047x over previous
import jax
import jax.numpy as jnp
from jax import lax
from jax.experimental import pallas as pl
from jax.experimental.pallas import tpu as pltpu
from jax.experimental.pallas import tpu_sc as plsc

L = 16
NC = 2
NS = 16
NW = NC * NS


def _eg_body(entt_ref, gatet_ref, eg_ref):
    et = entt_ref[...].T
    D = et.shape[1]
    eg_ref[:, :D] = et
    eg_ref[:, D:] = gatet_ref[...].T


def _eg_call(entt, gatet, bn=16384):
    D, N = entt.shape
    return pl.pallas_call(
        _eg_body,
        grid=((N + bn - 1) // bn,),
        in_specs=[
            pl.BlockSpec((D, bn), lambda i: (0, i)),
            pl.BlockSpec((D, bn), lambda i: (0, i)),
        ],
        out_specs=pl.BlockSpec((bn, 2 * D), lambda i: (i, 0)),
        out_shape=jax.ShapeDtypeStruct((N, 2 * D), jnp.float32),
    )(entt, gatet)


def _z_body(wordt_ref, z_ref):
    D = wordt_ref.shape[0]
    z_ref[:, :D] = wordt_ref[...].T


def _z_call(wordt, n, bn=16384):
    D = wordt.shape[0]
    return pl.pallas_call(
        _z_body,
        grid=((n + bn - 1) // bn,),
        in_specs=[pl.BlockSpec((D, bn), lambda i: (0, i))],
        out_specs=pl.BlockSpec((bn, 2 * D), lambda i: (i, 0)),
        out_shape=jax.ShapeDtypeStruct((n, 2 * D), jnp.float32),
    )(wordt)


def _make_sc_eg(B, D2):
    RPW = B // NW

    def body(head_hbm, tail_hbm, eg_hbm, heg_hbm, teg_hbm,
             hidx, tidx, hrows, trows, esem):
        wid = lax.axis_index("s") * NC + lax.axis_index("c")
        base = wid * RPW
        pltpu.sync_copy(head_hbm.at[pl.ds(base, RPW)], hidx)
        pltpu.sync_copy(tail_hbm.at[pl.ds(base, RPW)], tidx)
        cp_h = pltpu.async_copy(eg_hbm.at[hidx], hrows, esem)
        cp_t = pltpu.async_copy(eg_hbm.at[tidx], trows, esem)
        cp_h.wait()
        cp_t.wait()
        pltpu.sync_copy(hrows, heg_hbm.at[pl.ds(base, RPW)])
        pltpu.sync_copy(trows, teg_hbm.at[pl.ds(base, RPW)])

    return pl.kernel(
        body,
        out_type=[jax.ShapeDtypeStruct((B, D2), jnp.float32),
                  jax.ShapeDtypeStruct((B, D2), jnp.float32)],
        mesh=plsc.VectorSubcoreMesh(core_axis_name="c", subcore_axis_name="s"),
        compiler_params=pltpu.CompilerParams(use_tc_tiling_on_sc=False),
        scratch_types=[
            pltpu.VMEM((RPW,), jnp.int32),
            pltpu.VMEM((RPW,), jnp.int32),
            pltpu.VMEM((RPW, D2), jnp.float32),
            pltpu.VMEM((RPW, D2), jnp.float32),
            pltpu.SemaphoreType.DMA,
        ],
    )


def _make_sc_wordsum(B, D, W):
    RPW = B // NW
    SPC = 2
    IDXM = SPC * W
    CPW = RPW // SPC
    NBUF = 8

    def body(hd_hbm, td_hbm, word_hbm, wsum_hbm,
             widx, acc, *bufs_sems):
        bufs = bufs_sems[:NBUF]
        sems = bufs_sems[NBUF:NBUF + NBUF]
        wid = lax.axis_index("s") * NC + lax.axis_index("c")
        base = wid * RPW

        def word_phase(src_hbm, col0):
            pltpu.sync_copy(src_hbm.at[pl.ds(wid * CPW, CPW)], widx)
            for p in range(NBUF - 1):
                pltpu.async_copy(word_hbm.at[widx.at[p]], bufs[p], sems[p])

            def g_body(g, _):
                for b in range(NBUF):
                    c = g * NBUF + b
                    buf, sem = bufs[b], sems[b]
                    nb = (b + NBUF - 1) % NBUF
                    pltpu.make_async_copy(
                        word_hbm.at[widx.at[c]], buf, sem).wait()

                    @pl.when(c + NBUF - 1 < CPW)
                    def _issue_next():
                        pltpu.async_copy(
                            word_hbm.at[widx.at[c + NBUF - 1]],
                            bufs[nb], sems[nb])

                    for s in range(SPC):
                        def j_body(j, accs, _s=s):
                            return tuple(
                                accs[k] + buf[_s * W + j, pl.ds(k * L, L)]
                                for k in range(D // L))
                        accs = lax.fori_loop(
                            0, W, j_body,
                            tuple(jnp.zeros((L,), jnp.float32)
                                  for _ in range(D // L)))
                        row = SPC * c + s
                        for k in range(D // L):
                            acc[row, pl.ds(col0 + k * L, L)] = accs[k]
                return 0

            lax.fori_loop(0, CPW // NBUF, g_body, 0)

        word_phase(hd_hbm, 0)
        word_phase(td_hbm, D)
        pltpu.sync_copy(acc, wsum_hbm.at[pl.ds(base, RPW)])

    return pl.kernel(
        body,
        out_type=[jax.ShapeDtypeStruct((B, 2 * D), jnp.float32)],
        mesh=plsc.VectorSubcoreMesh(core_axis_name="c", subcore_axis_name="s"),
        compiler_params=pltpu.CompilerParams(use_tc_tiling_on_sc=False),
        scratch_types=[
            pltpu.VMEM((CPW, IDXM), jnp.int32),
            pltpu.VMEM((RPW, 2 * D), jnp.float32),
        ] + [pltpu.VMEM((IDXM, D), jnp.float32) for _ in range(NBUF)]
          + [pltpu.SemaphoreType.DMA for _ in range(NBUF)],
    )


def _dense_body(heg_ref, teg_ref, ws_ref, wh_ref, wt_ref, wb_ref,
                relt_ref, negt_ref, ot_ref):
    D = wh_ref.shape[1]
    heg = heg_ref[...]
    teg = teg_ref[...]
    ws = ws_ref[...]
    gh = jax.nn.sigmoid(heg[:, D:])
    gt = jax.nn.sigmoid(teg[:, D:])
    h = gh * heg[:, :D] + (1.0 - gh) * ws[:, :D]
    t = gt * teg[:, :D] + (1.0 - gt) * ws[:, D:]
    hrt = jnp.tanh(h * wh_ref[...] + t * wt_ref[...] + wb_ref[...])
    scores = lax.dot_general(relt_ref[...], hrt, (((0,), (1,)), ((), ())),
                             preferred_element_type=jnp.float32)
    w = jnp.abs(negt_ref[...])
    m = jnp.max(w * scores, axis=0, keepdims=True)
    e = jnp.exp(scores - m)
    s = jnp.sum(e * w, axis=0, keepdims=True)
    ot_ref[...] = e / s * w


def _dense_call(heg, teg, wsum, wh, wt, wb, relt, negt, bb=2048):
    B = heg.shape[0]
    D = wh.shape[1]
    R = negt.shape[0]
    return pl.pallas_call(
        _dense_body,
        grid=(B // bb,),
        in_specs=[
            pl.BlockSpec((bb, 2 * D), lambda i: (i, 0)),
            pl.BlockSpec((bb, 2 * D), lambda i: (i, 0)),
            pl.BlockSpec((bb, 2 * D), lambda i: (i, 0)),
            pl.BlockSpec((1, D), lambda i: (0, 0)),
            pl.BlockSpec((1, D), lambda i: (0, 0)),
            pl.BlockSpec((1, D), lambda i: (0, 0)),
            pl.BlockSpec((D, R), lambda i: (0, 0)),
            pl.BlockSpec((R, bb), lambda i: (0, i)),
        ],
        out_specs=pl.BlockSpec((R, bb), lambda i: (0, i)),
        out_shape=jax.ShapeDtypeStruct((R, B), jnp.float32),
    )(heg, teg, wsum, wh, wt, wb, relt, negt)


def kernel(triple, hd, td, neg_sample_r, entity_emb, relation_emb, word_emb,
           gate_emb, weight_h, weight_t, weight_bias):
    B, W = hd.shape
    D = entity_emb.shape[1]
    V = entity_emb.shape[0]
    head = triple[:, 0]
    tail = triple[:, 1]
    z = _z_call(word_emb.T, V)
    entt_b, gatet_b, z_b = lax.optimization_barrier(
        (entity_emb.T, gate_emb.T, z))
    word_lin = z_b.reshape(2 * V, D)
    hd2 = (hd * 2).reshape(B * W // (2 * W), 2 * W)
    td2 = (td * 2).reshape(B * W // (2 * W), 2 * W)
    (wsum,) = _make_sc_wordsum(B, D, W)(hd2, td2, word_lin)
    eg = _eg_call(entt_b, gatet_b)
    heg, teg = _make_sc_eg(B, 2 * D)(head, tail, eg)
    out_t = _dense_call(heg, teg, wsum, weight_h.reshape(1, D),
                        weight_t.reshape(1, D), weight_bias.reshape(1, D),
                        relation_emb.T, neg_sample_r.T)
    return out_t.T
